# baseline (speedup 1.0000x reference)
import math
from functools import partial

import jax
import jax.numpy as jnp
import numpy as np
from jax.experimental import pallas as pl
from jax.experimental.pallas import tpu as pltpu
from jax.sharding import Mesh, PartitionSpec as P

B, S, H, NH = 2, 2048, 2048, 16
HD = H // NH
SCALE = 1.0 / math.sqrt(HD)
LOG2E = math.log2(math.e)
NCORES = 2
S2 = S // NCORES

QKV_NG = 4
ATT_HG = 2
ATT_BQ = 256
OUT_MT = 512


def _q_kernel(x_ref, w_ref, o_ref):
    acc = jnp.dot(x_ref[0], w_ref[...], preferred_element_type=jnp.float32)
    for j in range(QKV_NG):
        o_ref[0, j] = acc[:, j * HD:(j + 1) * HD]


def _kv_kernel(x_ref, w_ref, o_ref):
    acc = jnp.dot(x_ref[0], w_ref[...], preferred_element_type=jnp.float32)
    for j in range(QKV_NG):
        o_ref[0, 0, j] = acc[:, j * HD:(j + 1) * HD]


def _attn_kernel(q_ref, k_ref, v_ref, pb_ref, w_ref, ctx_ref):
    for h in range(ATT_HG):
        pb2 = pb_ref[h] * LOG2E
        for b in range(B):
            q = q_ref[b, h] * (SCALE * LOG2E)
            k = k_ref[0, b, h]
            s = jax.lax.dot_general(q, k, (((1,), (1,)), ((), ())),
                                    preferred_element_type=jnp.float32)
            p = jnp.exp2(s + pb2)
            w = p * (1.0 / jnp.sum(p, axis=-1, keepdims=True))
            w_ref[b, h] = w
            ctx = jnp.dot(w, v_ref[0, b, h], preferred_element_type=jnp.float32)
            ctx_ref[b, :, h * HD:(h + 1) * HD] = ctx


def _out_kernel(x_ref, w_ref, o_ref):
    o_ref[0] = jnp.dot(x_ref[0], w_ref[...], preferred_element_type=jnp.float32)


def _shard_body(hs, pb, wqkv, wout):
    f32 = jnp.float32
    par = pltpu.CompilerParams(dimension_semantics=("arbitrary", "arbitrary"))

    q_arr = pl.pallas_call(
        _q_kernel,
        grid=(B, NH // QKV_NG),
        in_specs=[
            pl.BlockSpec((1, S2, H), lambda b, n: (b, 0, 0)),
            pl.BlockSpec((H, QKV_NG * HD), lambda b, n: (0, n)),
        ],
        out_specs=pl.BlockSpec((1, QKV_NG, S2, HD), lambda b, n: (b, n, 0, 0)),
        out_shape=jax.ShapeDtypeStruct((B, NH, S2, HD), f32),
        compiler_params=par,
    )(hs, wqkv)

    kv_local = pl.pallas_call(
        _kv_kernel,
        grid=(B, 2 * NH // QKV_NG),
        in_specs=[
            pl.BlockSpec((1, S2, H), lambda b, n: (b, 0, 0)),
            pl.BlockSpec((H, QKV_NG * HD),
                         lambda b, n: (0, n + NH // QKV_NG)),
        ],
        out_specs=pl.BlockSpec(
            (1, 1, QKV_NG, S2, HD),
            lambda b, n: (n * QKV_NG // NH, b, n % (NH // QKV_NG), 0, 0)),
        out_shape=jax.ShapeDtypeStruct((2, B, NH, S2, HD), f32),
        compiler_params=par,
    )(hs, wqkv)

    kv = jax.lax.all_gather(kv_local, "c", axis=3, tiled=True)

    n_hg = NH // ATT_HG
    n_q = S2 // ATT_BQ
    weights, context = pl.pallas_call(
        _attn_kernel,
        grid=(n_hg, n_q),
        in_specs=[
            pl.BlockSpec((B, ATT_HG, ATT_BQ, HD), lambda g, q: (0, g, q, 0)),
            pl.BlockSpec((1, B, ATT_HG, S, HD), lambda g, q: (0, 0, g, 0, 0)),
            pl.BlockSpec((1, B, ATT_HG, S, HD), lambda g, q: (1, 0, g, 0, 0)),
            pl.BlockSpec((ATT_HG, ATT_BQ, S), lambda g, q: (g, q, 0)),
        ],
        out_specs=[
            pl.BlockSpec((B, ATT_HG, ATT_BQ, S), lambda g, q: (0, g, q, 0)),
            pl.BlockSpec((B, ATT_BQ, ATT_HG * HD), lambda g, q: (0, q, g)),
        ],
        out_shape=[
            jax.ShapeDtypeStruct((B, NH, S2, S), f32),
            jax.ShapeDtypeStruct((B, S2, H), f32),
        ],
        compiler_params=par,
    )(q_arr, kv, kv, pb)

    attn_output = pl.pallas_call(
        _out_kernel,
        grid=(B, S2 // OUT_MT),
        in_specs=[
            pl.BlockSpec((1, OUT_MT, H), lambda b, m: (b, m, 0)),
            pl.BlockSpec((H, H), lambda b, m: (0, 0)),
        ],
        out_specs=pl.BlockSpec((1, OUT_MT, H), lambda b, m: (b, m, 0)),
        out_shape=jax.ShapeDtypeStruct((B, S2, H), f32),
        compiler_params=par,
    )(context, wout)

    return attn_output, weights


def kernel(hidden_states, position_bias, W_qkv, W_out):
    mesh = Mesh(np.array(jax.devices()[:NCORES]), ("c",))
    body = partial(jax.shard_map,
                   mesh=mesh,
                   in_specs=(P(None, "c", None), P(None, "c", None), P(), P()),
                   out_specs=(P(None, "c", None), P(None, None, "c", None)),
                   check_vma=False,
                   )(_shard_body)
    return body(hidden_states, position_bias, W_qkv, W_out)
